# speedup vs baseline: 3.6345x; 3.6345x over previous
import functools

import jax
import jax.numpy as jnp
from jax import lax
from jax.experimental import pallas as pl
from jax.experimental.pallas import tpu as pltpu
from jax.experimental.pallas import tpu_sc as plsc

N = 100000
D = 128
G = 512

CHUNK = 80
N_CHUNKS = N // CHUNK
NW = 32
MAX_ITERS = -(-N_CHUNKS // NW)

_mesh = plsc.VectorSubcoreMesh(core_axis_name="c", subcore_axis_name="s")


@functools.partial(
    pl.kernel,
    out_type=jax.ShapeDtypeStruct((2, G, D), jnp.float32),
    mesh=_mesh,
    scratch_types=[
        pltpu.VMEM((CHUNK,), jnp.int32),
        pltpu.VMEM((CHUNK, D), jnp.float32),
        pltpu.VMEM_SHARED((G, D), jnp.float32),
    ],
)
def _sc_segsum(nf_hbm, batch_hbm, zeros_hbm, part_hbm, idx_v, rows_v, acc):
    cid = lax.axis_index("c")
    sid = lax.axis_index("s")
    wid = sid * 2 + cid

    pltpu.sync_copy(zeros_hbm.at[pl.ds(sid * 32, 32)], acc.at[pl.ds(sid * 32, 32)])
    plsc.subcore_barrier()

    def body(i, carry):
        chunk = wid + NW * i

        @pl.when(chunk < N_CHUNKS)
        def _():
            off = chunk * CHUNK
            pltpu.sync_copy(batch_hbm.at[pl.ds(off, CHUNK)], idx_v)
            pltpu.sync_copy(nf_hbm.at[pl.ds(off, CHUNK), :], rows_v)
            pltpu.sync_copy(rows_v, acc.at[idx_v], add=True)

        return carry

    lax.fori_loop(0, MAX_ITERS, body, 0)
    plsc.subcore_barrier()

    pltpu.sync_copy(acc.at[pl.ds(sid * 32, 32)], part_hbm.at[cid, pl.ds(sid * 32, 32)])


def _tc_add(p_ref, o_ref):
    o_ref[...] = p_ref[0] + p_ref[1]


def kernel(node_features, batch):
    zeros = jnp.zeros((G, D), jnp.float32)
    partials = _sc_segsum(node_features, batch, zeros)
    return pl.pallas_call(
        _tc_add,
        out_shape=jax.ShapeDtypeStruct((G, D), jnp.float32),
    )(partials)


# speedup vs baseline: 7.2556x; 1.9963x over previous
import functools

import jax
import jax.numpy as jnp
from jax import lax
from jax.experimental import pallas as pl
from jax.experimental.pallas import tpu as pltpu
from jax.experimental.pallas import tpu_sc as plsc

N = 100000
D = 128
G = 512

CHUNK = 80
N_CHUNKS = N // CHUNK
NW = 32
K_STEPS = -(-N_CHUNKS // NW)
NBUF = 4

_mesh = plsc.VectorSubcoreMesh(core_axis_name="c", subcore_axis_name="s")


@functools.partial(
    pl.kernel,
    out_type=jax.ShapeDtypeStruct((2, G, D), jnp.float32),
    mesh=_mesh,
    scratch_types=[
        pltpu.VMEM((NBUF, CHUNK), jnp.int32),
        pltpu.VMEM((NBUF, CHUNK, D), jnp.float32),
        pltpu.VMEM_SHARED((G, D), jnp.float32),
        pltpu.SemaphoreType.DMA((NBUF,)),
    ],
)
def _sc_segsum(nf_hbm, batch2d_hbm, zeros_hbm, part_hbm,
               idx_v, rows_v, acc, gsem):
    cid = lax.axis_index("c")
    sid = lax.axis_index("s")
    wid = sid * 2 + cid

    pltpu.sync_copy(zeros_hbm.at[pl.ds(sid * 32, 32)], acc.at[pl.ds(sid * 32, 32)])
    plsc.subcore_barrier()

    def valid(k):
        return (wid + NW * k) < N_CHUNKS

    def gather_issue(k, b):
        c = wid + NW * k
        pltpu.async_copy(batch2d_hbm.at[c], idx_v.at[b], gsem.at[b])
        pltpu.async_copy(nf_hbm.at[pl.ds(c * CHUNK, CHUNK), :], rows_v.at[b],
                         gsem.at[b])

    def gather_wait(b):
        pltpu.make_async_copy(batch2d_hbm.at[0], idx_v.at[b], gsem.at[b]).wait()
        pltpu.make_async_copy(nf_hbm.at[pl.ds(0, CHUNK), :], rows_v.at[b],
                              gsem.at[b]).wait()

    for j in range(NBUF):
        @pl.when(valid(j))
        def _(j=j):
            gather_issue(j, j)

    def body(g, carry):
        for b in range(NBUF):
            k = NBUF * g + b

            @pl.when(valid(k))
            def _():
                gather_wait(b)
                pltpu.sync_copy(rows_v.at[b], acc.at[idx_v.at[b]], add=True)

            j = k + NBUF
            @pl.when((j < K_STEPS) & valid(j))
            def _():
                gather_issue(j, b)

        return carry

    lax.fori_loop(0, K_STEPS // NBUF, body, 0)
    plsc.subcore_barrier()

    pltpu.sync_copy(acc.at[pl.ds(sid * 32, 32)], part_hbm.at[cid, pl.ds(sid * 32, 32)])


def _tc_add(p_ref, o_ref):
    o_ref[...] = p_ref[0] + p_ref[1]


def kernel(node_features, batch):
    zeros = jnp.zeros((G, D), jnp.float32)
    batch2d = batch.reshape(N_CHUNKS, CHUNK)
    partials = _sc_segsum(node_features, batch2d, zeros)
    return pl.pallas_call(
        _tc_add,
        out_shape=jax.ShapeDtypeStruct((G, D), jnp.float32),
    )(partials)
